# Initial kernel scaffold; baseline (speedup 1.0000x reference)
import jax
import jax.numpy as jnp
from jax.experimental import pallas as pl


def kernel(x, edge_index, edge_weight, conv_w, conv_b, gcn_w, gcn_b, bn_gamma, bn_beta):
    raise NotImplementedError("write your pallas kernel here")


# speedup vs baseline: 11.8645x; 11.8645x over previous
import functools

import jax
import jax.numpy as jnp
from jax import lax
from jax.experimental import pallas as pl
from jax.experimental.pallas import tpu as pltpu
from jax.experimental.pallas import tpu_sc as plsc

T, N, C = 12, 10000, 128
NB = 2000
NT = 16
NSC = 2
B = 128
NN = 10240
NPT = NN // NT
NDEG = 10240
TPC = T // NSC
F32 = jnp.float32
I32 = jnp.int32


def _deg_body(nbat, cols_hbm, ews_hbm, deg_hbm, col_l, ew_l, dbuf, deg_sh):
    c = lax.axis_index("c")
    s = lax.axis_index("s")
    pltpu.sync_copy(cols_hbm.at[s], col_l)
    pltpu.sync_copy(ews_hbm.at[s], ew_l)

    z16 = jnp.zeros((16,), F32)
    for k in range(NPT // 16):
        dbuf[pl.ds(k * 16, 16)] = z16
    pltpu.sync_copy(dbuf, deg_sh.at[pl.ds(s * NPT, NPT)])
    plsc.subcore_barrier()

    def _deg(b, carry):
        pltpu.sync_copy(ew_l.at[b], deg_sh.at[col_l.at[b]], add=True)
        return carry
    lax.fori_loop(0, nbat, _deg, None)
    plsc.subcore_barrier()

    @pl.when(c == 0)
    def _():
        pltpu.sync_copy(deg_sh.at[pl.ds(s * NPT, NPT)], dbuf)
        pltpu.sync_copy(dbuf, deg_hbm.at[pl.ds(s * NPT, NPT)])


def _run_deg(cols3, ews3, nbat):
    mesh = plsc.VectorSubcoreMesh(core_axis_name="c", subcore_axis_name="s")
    k = pl.kernel(
        functools.partial(_deg_body, nbat),
        out_type=jax.ShapeDtypeStruct((NDEG,), F32),
        mesh=mesh,
        scratch_types=[
            pltpu.VMEM((nbat, B), I32),
            pltpu.VMEM((nbat, B), F32),
            pltpu.VMEM((NPT,), F32),
            pltpu.VMEM_SHARED((NDEG,), F32),
        ],
    )
    return k(cols3, ews3)


def _h_body(xm_ref, x0_ref, xp_ref, cw_ref, gw_ref, cb_ref, deg_ref,
            h_ref, dv_ref):
    t = pl.program_id(0)
    dn = (((1,), (1,)), ((), ()))
    y = lax.dot_general(x0_ref[0], cw_ref[1], dn, preferred_element_type=F32)
    ym = lax.dot_general(xm_ref[0], cw_ref[0], dn, preferred_element_type=F32)
    yp = lax.dot_general(xp_ref[0], cw_ref[2], dn, preferred_element_type=F32)
    mprev = (t > 0).astype(F32)
    mnext = (t < T - 1).astype(F32)
    y = y + mprev * ym + mnext * yp + cb_ref[...]
    h = lax.dot_general(y, gw_ref[...], dn, preferred_element_type=F32)
    dv = lax.rsqrt(deg_ref[...] + 1.0)
    h_ref[...] = h * dv
    dv_ref[...] = dv


def _make_h(x, cw3, gw, cb2, deg2):
    grid = (T, N // NB)
    return pl.pallas_call(
        _h_body,
        grid=grid,
        in_specs=[
            pl.BlockSpec((1, NB, C), lambda t, n: (jnp.maximum(t - 1, 0), n, 0)),
            pl.BlockSpec((1, NB, C), lambda t, n: (t, n, 0)),
            pl.BlockSpec((1, NB, C), lambda t, n: (jnp.minimum(t + 1, T - 1), n, 0)),
            pl.BlockSpec((3, C, C), lambda t, n: (0, 0, 0)),
            pl.BlockSpec((C, C), lambda t, n: (0, 0)),
            pl.BlockSpec((1, C), lambda t, n: (0, 0)),
            pl.BlockSpec((NB, 1), lambda t, n: (n, 0)),
        ],
        out_specs=[
            pl.BlockSpec((NB, C), lambda t, n: (t * (N // NB) + n, 0)),
            pl.BlockSpec((NB, 1), lambda t, n: (n, 0)),
        ],
        out_shape=[
            jax.ShapeDtypeStruct((T * N, C), F32),
            jax.ShapeDtypeStruct((N, 1), F32),
        ],
    )(x, x, x, cw3, gw, cb2, deg2)


CH = 16


def _sc_body(nbat, h_hbm, edata_hbm, s_hbm,
             ebufa, ebufb, bufa, bufb, acc, sema, semb, semea, semeb):
    c = lax.axis_index("c")
    s = lax.axis_index("s")
    nchunk = nbat // CH

    z16 = jnp.zeros((16,), F32)

    def _zb(i, carry):
        for k in range(C // 16):
            bufa[i, pl.ds(k * 16, 16)] = z16
        return carry
    lax.fori_loop(0, B, _zb, None)
    for k in range(NPT // B):
        pltpu.sync_copy(bufa, acc.at[pl.ds(s * NPT + k * B, B)])
    plsc.subcore_barrier()

    def _process_batch(buf, sem, ebuf, ib):
        pltpu.make_async_copy(h_hbm.at[ebuf.at[ib, 0]], buf, sem).wait()

        def _sc16(kk, carry):
            fvec = lax.bitcast_convert_type(ebuf[ib, 2, pl.ds(kk * 16, 16)], F32)
            for j in range(16):
                fj = fvec[j]
                for k2 in range(C // 16):
                    sl = pl.ds(k2 * 16, 16)
                    buf[kk * 16 + j, sl] = buf[kk * 16 + j, sl] * fj
            return carry
        lax.fori_loop(0, B // 16, _sc16, None)
        pltpu.sync_copy(buf, acc.at[ebuf.at[ib, 1]], add=True)

    def _tstep(ti, carry):
        tg = c * TPC + ti
        trow = tg * NN
        pltpu.sync_copy(edata_hbm.at[tg, s, pl.ds(0, CH)], ebufa)
        pltpu.async_copy(edata_hbm.at[tg, s, pl.ds(CH, CH)], ebufb, semeb)
        pltpu.async_copy(h_hbm.at[ebufa.at[0, 0]], bufa, sema)
        pltpu.async_copy(h_hbm.at[ebufa.at[1, 0]], bufb, semb)

        def _upair(u, carry2):
            notlast = u < nchunk // 2 - 1
            for half in (0, 1):
                ebuf = (ebufa, ebufb)[half]
                eoth = (ebufb, ebufa)[half]
                esem_oth = (semeb, semea)[half]
                esem_own = (semea, semeb)[half]

                def _pair(p, carry3, ebuf=ebuf):
                    for buf, sem, ib in ((bufa, sema, 2 * p),
                                         (bufb, semb, 2 * p + 1)):
                        _process_batch(buf, sem, ebuf, ib)
                        pltpu.async_copy(h_hbm.at[ebuf.at[ib + 2, 0]], buf, sem)
                    return carry3
                lax.fori_loop(0, CH // 2 - 1, _pair, None)

                def _last_pair_go(ebuf=ebuf, eoth=eoth, esem_oth=esem_oth):
                    pltpu.make_async_copy(
                        edata_hbm.at[tg, s, pl.ds(0, CH)], eoth,
                        esem_oth).wait()
                    _process_batch(bufa, sema, ebuf, CH - 2)
                    pltpu.async_copy(h_hbm.at[eoth.at[0, 0]], bufa, sema)
                    _process_batch(bufb, semb, ebuf, CH - 1)
                    pltpu.async_copy(h_hbm.at[eoth.at[1, 0]], bufb, semb)

                if half == 0:
                    _last_pair_go()
                else:
                    pl.when(notlast)(_last_pair_go)

                    @pl.when(jnp.logical_not(notlast))
                    def _(ebuf=ebuf):
                        _process_batch(bufa, sema, ebuf, CH - 2)
                        _process_batch(bufb, semb, ebuf, CH - 1)

                @pl.when(notlast)
                def _(ebuf=ebuf, esem_own=esem_own, half=half):
                    nxt = (2 * u + 2 + half) * CH
                    pltpu.async_copy(edata_hbm.at[tg, s, pl.ds(nxt, CH)],
                                     ebuf, esem_own)
            return carry2
        lax.fori_loop(0, nchunk // 2, _upair, None)

        plsc.subcore_barrier()
        def _zb2(i, carry2):
            for k in range(C // 16):
                bufb[i, pl.ds(k * 16, 16)] = z16
            return carry2
        lax.fori_loop(0, B, _zb2, None)
        for k in range(NPT // B):
            off = s * NPT + k * B
            pltpu.sync_copy(acc.at[pl.ds(off, B)], bufa)
            pltpu.sync_copy(bufa, s_hbm.at[pl.ds(trow + off, B)])
            pltpu.sync_copy(bufb, acc.at[pl.ds(off, B)])
        plsc.subcore_barrier()
        return carry
    lax.fori_loop(0, TPC, _tstep, None)


def _run_sc(h_flat, edata, nbat):
    mesh = plsc.VectorSubcoreMesh(core_axis_name="c", subcore_axis_name="s")
    k = pl.kernel(
        functools.partial(_sc_body, nbat),
        out_type=jax.ShapeDtypeStruct((T * NN, C), F32),
        mesh=mesh,
        scratch_types=[
            pltpu.VMEM((CH, 3, B), I32),
            pltpu.VMEM((CH, 3, B), I32),
            pltpu.VMEM((B, C), F32),
            pltpu.VMEM((B, C), F32),
            pltpu.VMEM_SHARED((NN, C), F32),
            pltpu.SemaphoreType.DMA,
            pltpu.SemaphoreType.DMA,
            pltpu.SemaphoreType.DMA,
            pltpu.SemaphoreType.DMA,
        ],
    )
    return k(h_flat, edata)


def _stat_body(s_ref, h_ref, dv_ref, b_ref, st_ref):
    nb = pl.program_id(1)
    dv = dv_ref[...]
    agg = dv * (s_ref[...] + h_ref[...]) + b_ref[...]

    @pl.when(nb == 0)
    def _():
        st_ref[...] = jnp.zeros_like(st_ref)
    st_ref[0, 0:1, :] += jnp.sum(agg, axis=0, keepdims=True)
    st_ref[0, 1:2, :] += jnp.sum(agg * agg, axis=0, keepdims=True)


def _bn_body(s_ref, h_ref, dv_ref, b_ref, st_ref, g_ref, be_ref, o_ref):
    dv = dv_ref[...]
    agg = dv * (s_ref[...] + h_ref[...]) + b_ref[...]
    mean = st_ref[0, 0:1, :] / N
    var = st_ref[0, 1:2, :] / N - mean * mean
    inv = lax.rsqrt(var + 1e-5)
    o_ref[...] = jnp.maximum((agg - mean) * inv * g_ref[...] + be_ref[...], 0.0)


def _run_epilogue(s_flat, h_flat, dinv2, gb2, gamma2, beta2):
    grid = (T, N // NB)
    sh = pl.BlockSpec((NB, C), lambda t, n: (t * (N // NB) + n, 0))
    dvs = pl.BlockSpec((NB, 1), lambda t, n: (n, 0))
    row = pl.BlockSpec((1, C), lambda t, n: (0, 0))
    sts = pl.BlockSpec((1, 2, C), lambda t, n: (t, 0, 0))
    stats = pl.pallas_call(
        _stat_body, grid=grid,
        in_specs=[sh, sh, dvs, row],
        out_specs=sts,
        out_shape=jax.ShapeDtypeStruct((T, 2, C), F32),
    )(s_flat, h_flat, dinv2, gb2)
    return pl.pallas_call(
        _bn_body, grid=grid,
        in_specs=[sh, sh, dvs, row, sts, row, row],
        out_specs=sh,
        out_shape=jax.ShapeDtypeStruct((T * N, C), F32),
    )(s_flat, h_flat, dinv2, gb2, stats, gamma2, beta2)


def kernel(x, edge_index, edge_weight, conv_w, conv_b, gcn_w, gcn_b,
           bn_gamma, bn_beta):
    ei = edge_index.astype(I32)
    e = ei.shape[1]
    nbat = -(-e // (NT * B))
    nbat = -(-nbat // (2 * CH)) * (2 * CH)
    epad = NT * nbat * B

    rowp = jnp.pad(ei[0], (0, epad - e)).reshape(NT, nbat, B)
    colp = jnp.pad(ei[1], (0, epad - e)).reshape(NT, nbat, B)
    ewp = jnp.pad(edge_weight, (0, epad - e)).reshape(NT, nbat, B)
    rows_t = rowp[None] + (jnp.arange(T, dtype=I32) * N)[:, None, None, None]
    ewbits = lax.bitcast_convert_type(ewp, I32)
    edata = jnp.stack(
        [rows_t,
         jnp.broadcast_to(colp[None], (T, NT, nbat, B)),
         jnp.broadcast_to(ewbits[None], (T, NT, nbat, B))], axis=3)

    deg = _run_deg(colp, ewp, nbat)
    cw3 = jnp.transpose(conv_w, (2, 0, 1))
    h_flat, dinv2 = _make_h(x, cw3, gcn_w, conv_b.reshape(1, C),
                            deg[:N].reshape(N, 1))
    s_pad = _run_sc(h_flat, edata, nbat)
    s_flat = s_pad.reshape(T, NN, C)[:, :N, :].reshape(T * N, C)
    out = _run_epilogue(s_flat, h_flat, dinv2, gcn_b.reshape(1, C),
                        bn_gamma.reshape(1, C), bn_beta.reshape(1, C))
    return out.reshape(T, N, C)
